# baseline (speedup 1.0000x reference)
import dataclasses

import jax
import jax.numpy as jnp
from jax import lax
from jax.experimental import pallas as pl
from jax.experimental.pallas import tpu as pltpu
from jax.experimental.pallas import tpu_sc as plsc

N_USERS = 1_000_000
N_NOTES = 100_000
F = 16
B = 16384
NC = 2
NS = 16
L = 16
NW = NC * NS
BPW = B // NW
CHUNK = 128
NCHUNK = BPW // CHUNK
NGROUP = BPW // L


def _mf_kernel(uidx_hbm, nidx_hbm, uf_hbm, nf_hbm, ui_hbm, ni_hbm, g_hbm,
               out_hbm,
               uidx_v, nidx_v, urow_v, nrow_v,
               uf_v, nf_v, ui_rows_v, ni_rows_v, out_v,
               g_v, sem):
    wid = lax.axis_index("s") * NC + lax.axis_index("c")

    pltpu.sync_copy(uidx_hbm.at[wid], uidx_v)
    pltpu.sync_copy(nidx_hbm.at[wid], nidx_v)
    pltpu.sync_copy(g_hbm, g_v.at[pl.ds(0, 1)])

    @pl.loop(0, NCHUNK)
    def _(c):
        @pl.loop(0, CHUNK // L)
        def _(k):
            sl = pl.ds(k * L, L)
            urow_v[c, sl] = jax.lax.shift_right_logical(uidx_v[c, sl], 4)
            nrow_v[c, sl] = jax.lax.shift_right_logical(nidx_v[c, sl], 4)

    copies = []
    for c in range(NCHUNK):
        dst = pl.ds(c * CHUNK, CHUNK)
        copies.append(pltpu.async_copy(uf_hbm.at[uidx_v.at[c]], uf_v.at[dst], sem))
        copies.append(pltpu.async_copy(nf_hbm.at[nidx_v.at[c]], nf_v.at[dst], sem))
        copies.append(pltpu.async_copy(ui_hbm.at[urow_v.at[c]], ui_rows_v.at[dst], sem))
        copies.append(pltpu.async_copy(ni_hbm.at[nrow_v.at[c]], ni_rows_v.at[dst], sem))
    for cp in copies:
        cp.wait()

    lane_iota = lax.iota(jnp.int32, L)
    gint = g_v[pl.ds(0, L)][0]

    @pl.loop(0, NGROUP)
    def _(g):
        base = g * L
        riota = lane_iota + base
        c = g // (CHUNK // L)
        off = (g % (CHUNK // L)) * L
        ulane = jnp.bitwise_and(uidx_v[c, pl.ds(off, L)], L - 1)
        nlane = jnp.bitwise_and(nidx_v[c, pl.ds(off, L)], L - 1)
        acc = (plsc.load_gather(ui_rows_v, [riota, ulane])
               + plsc.load_gather(ni_rows_v, [riota, nlane])
               + gint)
        for f in range(F):
            fvec = jnp.full((L,), f, jnp.int32)
            tu = plsc.load_gather(uf_v, [riota, fvec])
            tn = plsc.load_gather(nf_v, [riota, fvec])
            acc = acc + tu * tn
        out_v[pl.ds(base, L)] = acc

    pltpu.sync_copy(out_v, out_hbm.at[pl.ds(wid * BPW, BPW)])


@jax.jit
def kernel(user_indexes, note_indexes, user_factors, note_factors,
           user_intercepts, note_intercepts, global_intercept):
    mesh = plsc.VectorSubcoreMesh(core_axis_name="c", subcore_axis_name="s",
                                  num_cores=NC, num_subcores=NS)
    cp = pltpu.CompilerParams(use_tc_tiling_on_sc=False)
    if "needs_layout_passes" in pltpu.CompilerParams.__dataclass_fields__:
        cp = dataclasses.replace(cp, needs_layout_passes=False)
    kfn = pl.kernel(
        _mf_kernel,
        out_type=jax.ShapeDtypeStruct((B,), jnp.float32),
        mesh=mesh,
        compiler_params=cp,
        scratch_types=[
            pltpu.VMEM((NCHUNK, CHUNK), jnp.int32),
            pltpu.VMEM((NCHUNK, CHUNK), jnp.int32),
            pltpu.VMEM((NCHUNK, CHUNK), jnp.int32),
            pltpu.VMEM((NCHUNK, CHUNK), jnp.int32),
            pltpu.VMEM((BPW, F), jnp.float32),
            pltpu.VMEM((BPW, F), jnp.float32),
            pltpu.VMEM((BPW, L), jnp.float32),
            pltpu.VMEM((BPW, L), jnp.float32),
            pltpu.VMEM((BPW,), jnp.float32),
            pltpu.VMEM((L,), jnp.float32),
            pltpu.SemaphoreType.DMA,
        ],
    )
    return kfn(
        user_indexes.reshape(NW, NCHUNK, CHUNK),
        note_indexes.reshape(NW, NCHUNK, CHUNK),
        user_factors,
        note_factors,
        user_intercepts.reshape(N_USERS // L, L),
        note_intercepts.reshape(N_NOTES // L, L),
        global_intercept.reshape(1),
    )
